# baseline (speedup 1.0000x reference)
import functools

import numpy as np
import jax
import jax.numpy as jnp
from jax import lax
from jax.experimental import pallas as pl
from jax.experimental.pallas import tpu as pltpu
from jax.experimental.pallas import tpu_sc as plsc

B = 128
N = 100000
L = 16
RG = 8
RW = 4
TS = 32
NT = N - TS
CHUNKC = 4096
NFULL = 24
TC0 = NT - NFULL * CHUNKC
U = 8
WCLAMP = (N // 128 - 1) * 128

_info = plsc.get_sparse_core_info()
NC, NS = _info.num_cores, _info.num_subcores
NW = NC * NS
NG = B // RG

_HARM = np.concatenate([
    np.zeros(1, np.float32),
    np.cumsum((1.0 / np.arange(1, N, dtype=np.float32)), dtype=np.float32),
]).astype(np.float32)


def _gather(vec, idx):
    return vec.at[idx].get(mode="promise_in_bounds")


def _make_kernel():
    mesh = plsc.VectorSubcoreMesh(core_axis_name="c", subcore_axis_name="s")

    @functools.partial(
        pl.kernel,
        mesh=mesh,
        compiler_params=pltpu.CompilerParams(needs_layout_passes=False),
        name="wrpl_sc",
        out_type=jax.ShapeDtypeStruct((NW, L), jnp.float32),
        scratch_types=[
            pltpu.VMEM((RG, CHUNKC), jnp.float32),
            pltpu.VMEM((RG, CHUNKC), jnp.float32),
            pltpu.VMEM((RG, TC0), jnp.float32),
            pltpu.VMEM((RG * TS,), jnp.float32),
            pltpu.VMEM((B,), jnp.int32),
            pltpu.VMEM((RW, RG, 128), jnp.float32),
            pltpu.VMEM((RW, L), jnp.float32),
            pltpu.VMEM((RW, L), jnp.int32),
            pltpu.VMEM((RW, L), jnp.float32),
            pltpu.SMEM((RW,), jnp.int32),
            pltpu.VMEM((RW, L), jnp.float32),
            pltpu.VMEM((L,), jnp.float32),
            pltpu.SemaphoreType.DMA,
            pltpu.SemaphoreType.DMA,
            pltpu.SemaphoreType.DMA,
        ],
    )
    def wrpl(score_hbm, tail_hbm, target_hbm, harm_hbm, out_hbm,
             buf0, buf1, tbig, tbuf, tgt_v, gtw, gtr, accc, acch,
             bss, hwin, ov, sem0, sem1, semg):
        c = lax.axis_index("c")
        s = lax.axis_index("s")
        wid = c * NS + s
        g = wid // 2
        h = wid % 2
        g8 = pl.multiple_of(g * RG, 8)
        r0 = h * RW
        iota = lax.iota(jnp.int32, L)

        bufs = (buf0, buf1)
        sems = (sem0, sem1)

        def issue_full(k, slot):
            col = pl.multiple_of(k * CHUNKC, 128)
            return pltpu.async_copy(
                score_hbm.at[pl.ds(g8, RG), pl.ds(col, CHUNKC)],
                bufs[slot], sems[slot])

        handles = [issue_full(0, 0), issue_full(1, 1)]

        pltpu.sync_copy(target_hbm, tgt_v)
        pltpu.sync_copy(tail_hbm.at[pl.ds(g8 * TS, RG * TS)], tbuf)

        tvs, wbs = [], []
        ghandles = []
        for q in range(RW):
            b = g8 + r0 + q
            bb = (b // L) * L
            twin = tgt_v[pl.ds(bb, L)]
            tv = _gather(twin, jnp.zeros((L,), jnp.int32) + (b - bb))
            t_s = jnp.max(tv)
            wb = pl.multiple_of(
                jnp.minimum((t_s // 128) * 128, WCLAMP), 128)
            ghandles.append(pltpu.async_copy(
                score_hbm.at[pl.ds(g8, RG), pl.ds(wb, 128)],
                gtw.at[q], semg))
            tvs.append(tv)
            wbs.append(wb)
        for hd in ghandles:
            hd.wait()

        for q in range(RW):
            tv, wb = tvs[q], wbs[q]
            t_s = jnp.max(tv)
            bs = (t_s // L) * L
            lane = tv - bs
            bo_a = jnp.minimum(bs - wb, 128 - L)
            bo_a = jnp.maximum(bo_a, 0)
            v16a0 = gtw[q, q, pl.ds(bo_a, L)]
            v16a1 = gtw[q, RW + q, pl.ds(bo_a, L)]
            v16a = jnp.where(h == 0, v16a0, v16a1)
            bo_b = jnp.clip(bs - NT, 0, TS - L)
            v16b = tbuf[pl.ds((r0 + q) * TS + bo_b, L)]
            v16 = jnp.where(t_s < NT, v16a, v16b)
            gtv = _gather(v16, lane)
            ctb = plsc.all_reduce_population_count(
                (v16 == gtv) & (iota < lane))
            accc[q] = ctb
            acch[q] = jnp.where(iota == 0, -1.0, 0.0)
            gtr[q] = gtv
            bss[q] = bs

        def process(loader, base_col, nv, unroll):
            def row_body(q, _):
                rr8 = r0 + q
                gtv = gtr[q]
                gtm1 = gtv - 1.0
                bs = bss[q]
                zi = jnp.zeros((L,), jnp.int32)
                zf = jnp.zeros((L,), jnp.float32)
                accs = tuple((zi, zf) for _ in range(unroll))

                def body(i, carry):
                    out = []
                    for u in range(unroll):
                        cg, hh = carry[u]
                        j = base_col + (i + u) * L
                        v = loader(rr8, i + u)
                        mgt = v > gtv
                        mge = v >= gtv
                        m = jnp.where(j < bs, mge, mgt)
                        cg = cg + plsc.all_reduce_population_count(m)
                        hh = hh + jnp.maximum(v - gtm1, 0.0)
                        out.append((cg, hh))
                    return tuple(out)

                accs = plsc.parallel_loop(0, nv, unroll, carry=accs)(body)
                cnt = accc[q]
                hht = acch[q]
                for (cg, hh) in accs:
                    cnt = cnt + cg
                    hht = hht + hh
                accc[q] = cnt
                acch[q] = hht
                return 0

            lax.fori_loop(0, RW, row_body, 0)

        def buf_loader(buf):
            def load(rr8, t):
                return buf[rr8, pl.ds(t * L, L)]
            return load

        for k in range(NFULL):
            slot = k % 2
            handles[slot].wait()
            process(buf_loader(bufs[slot]), k * CHUNKC, CHUNKC // L, U)
            if k + 2 < NFULL:
                handles[slot] = issue_full(k + 2, slot)

        pltpu.sync_copy(
            score_hbm.at[pl.ds(g8, RG),
                         pl.ds(pl.multiple_of(NFULL * CHUNKC, 128), TC0)],
            tbig)
        process(buf_loader(tbig), NFULL * CHUNKC, TC0 // L, U)

        def tail_load(rr8, t):
            return tbuf[pl.ds(rr8 * TS + t * L, L)]
        process(tail_load, NT, TS // L, TS // L)

        ris, hsums, hoffs = [], [], []
        hhandles = []
        for q in range(RW):
            ri = jnp.max(accc[q])
            hsum = jnp.sum(acch[q])
            hoff = pl.multiple_of(jnp.minimum((ri // 8) * 8, N - L), 8)
            hhandles.append(pltpu.async_copy(
                harm_hbm.at[pl.ds(hoff, L)], hwin.at[q], semg))
            ris.append(ri)
            hsums.append(hsum)
            hoffs.append(hoff)
        for hd in hhandles:
            hd.wait()

        contrib = jnp.zeros((L,), jnp.float32)
        for q in range(RW):
            riv = jnp.zeros((L,), jnp.int32) + ris[q]
            hv = _gather(hwin[q], riv - hoffs[q])
            wv = jnp.where(riv > 0,
                           hv / jnp.maximum(riv.astype(jnp.float32), 1.0),
                           0.0)
            contrib = contrib + jnp.where(iota == q,
                                          wv * (hsums[q] * (1.0 / B)), 0.0)
        ov[...] = contrib
        pltpu.sync_copy(ov, out_hbm.at[wid])

    return wrpl


_WRPL = _make_kernel()


def kernel(score, target):
    tail = score[:, NT:].reshape(-1)
    parts = _WRPL(score, tail, target, jnp.asarray(_HARM))
    return jnp.sum(parts)


# speedup vs baseline: 1.0965x; 1.0965x over previous
import functools

import numpy as np
import jax
import jax.numpy as jnp
from jax import lax
from jax.experimental import pallas as pl
from jax.experimental.pallas import tpu as pltpu
from jax.experimental.pallas import tpu_sc as plsc

B = 128
N = 100000
L = 16
RG = 8
TS = 32
NT = N - TS
SPLIT = 50048
CHUNKC = 4096
NFULL = 12
T0 = SPLIT - NFULL * CHUNKC
T1 = NT - SPLIT - NFULL * CHUNKC
U = 8
UT0 = 8
UT1 = 8
WCLAMP = (N // 128 - 1) * 128

_info = plsc.get_sparse_core_info()
NC, NS = _info.num_cores, _info.num_subcores
NW = NC * NS
NG = B // RG

_HARM = np.concatenate([
    np.zeros(1, np.float32),
    np.cumsum((1.0 / np.arange(1, N, dtype=np.float32)), dtype=np.float32),
]).astype(np.float32)


def _gather(vec, idx):
    return vec.at[idx].get(mode="promise_in_bounds")


def _make_kernel():
    mesh = plsc.VectorSubcoreMesh(core_axis_name="c", subcore_axis_name="s")

    @functools.partial(
        pl.kernel,
        mesh=mesh,
        compiler_params=pltpu.CompilerParams(needs_layout_passes=False),
        name="wrpl_sc",
        out_type=jax.ShapeDtypeStruct((NW, L), jnp.float32),
        scratch_types=[
            pltpu.VMEM((RG, CHUNKC), jnp.float32),
            pltpu.VMEM((RG, CHUNKC), jnp.float32),
            pltpu.VMEM((RG, T0), jnp.float32),
            pltpu.VMEM((RG, T1), jnp.float32),
            pltpu.VMEM((RG * TS,), jnp.float32),
            pltpu.VMEM((B,), jnp.int32),
            pltpu.VMEM((RG, RG, 128), jnp.float32),
            pltpu.VMEM((RG, L), jnp.float32),
            pltpu.VMEM((RG, L), jnp.int32),
            pltpu.VMEM((RG, L), jnp.float32),
            pltpu.SMEM((RG,), jnp.int32),
            pltpu.VMEM((RG, L), jnp.float32),
            pltpu.VMEM((L,), jnp.float32),
            pltpu.VMEM((L,), jnp.float32),
            pltpu.VMEM_SHARED((NS, 128), jnp.float32),
            pltpu.SemaphoreType.DMA,
            pltpu.SemaphoreType.DMA,
            pltpu.SemaphoreType.DMA,
        ],
    )
    def wrpl(score_hbm, tail_hbm, target_hbm, harm_hbm, out_hbm,
             buf0, buf1, tail0, tail1, tbuf, tgt_v, gtw, gtr, accc, acch,
             bss, hwin, ov, pv, shared, sem0, sem1, semg):
        c = lax.axis_index("c")
        s = lax.axis_index("s")
        g = c * (NG // NC) + s // 2
        h = s % 2
        g8 = pl.multiple_of(g * RG, 8)
        c0 = h * SPLIT
        chw = SPLIT + h * (N - 2 * SPLIT)
        iota = lax.iota(jnp.int32, L)

        bufs = (buf0, buf1)
        sems = (sem0, sem1)

        def issue_full(k, slot):
            col = pl.multiple_of(h * SPLIT + k * CHUNKC, 128)
            return pltpu.async_copy(
                score_hbm.at[pl.ds(g8, RG), pl.ds(col, CHUNKC)],
                bufs[slot], sems[slot])

        handles = [issue_full(0, 0), issue_full(1, 1)]

        pltpu.sync_copy(target_hbm, tgt_v)
        pltpu.sync_copy(tail_hbm.at[pl.ds(g8 * TS, RG * TS)], tbuf)

        tvs, wbs = [], []
        ghandles = []
        for rr in range(RG):
            b = g8 + rr
            bb = (b // L) * L
            twin = tgt_v[pl.ds(bb, L)]
            tv = _gather(twin, jnp.zeros((L,), jnp.int32) + (b - bb))
            t_s = jnp.max(tv)
            wb = pl.multiple_of(
                jnp.minimum((t_s // 128) * 128, WCLAMP), 128)
            ghandles.append(pltpu.async_copy(
                score_hbm.at[pl.ds(g8, RG), pl.ds(wb, 128)],
                gtw.at[rr], semg))
            tvs.append(tv)
            wbs.append(wb)
        for hd in ghandles:
            hd.wait()

        for rr in range(RG):
            tv, wb = tvs[rr], wbs[rr]
            t_s = jnp.max(tv)
            bs = (t_s // L) * L
            lane = tv - bs
            bo_a = jnp.minimum(bs - wb, 128 - L)
            bo_a = jnp.maximum(bo_a, 0)
            v16a = gtw[rr, rr, pl.ds(bo_a, L)]
            bo_b = jnp.clip(bs - NT, 0, TS - L)
            v16b = tbuf[pl.ds(rr * TS + bo_b, L)]
            v16 = jnp.where(t_s < NT, v16a, v16b)
            gtv = _gather(v16, lane)
            ctb = plsc.all_reduce_population_count(
                (v16 == gtv) & (iota < lane))
            in_half_bs = (bs >= c0) & (bs - c0 < chw)
            in_half_t = (t_s >= c0) & (t_s - c0 < chw)
            accc[rr] = jnp.where(in_half_bs, ctb, 0)
            acch[rr] = jnp.where(iota == 0,
                                 jnp.where(in_half_t, -1.0, 0.0), 0.0)
            gtr[rr] = gtv
            bss[rr] = bs

        def process(loader, nv, unroll):
            def row_body(rr, _):
                gtv = gtr[rr]
                gtm1 = gtv - 1.0
                bs = bss[rr]
                zi = jnp.zeros((L,), jnp.int32)
                zf = jnp.zeros((L,), jnp.float32)
                accs = tuple((zi, zf) for _ in range(unroll))

                def body(i, carry):
                    out = []
                    for u in range(unroll):
                        cg, hh = carry[u]
                        v, j = loader(rr, i + u)
                        mgt = v > gtv
                        mge = v >= gtv
                        m = jnp.where(j < bs, mge, mgt)
                        cg = cg + plsc.all_reduce_population_count(m)
                        hh = hh + jnp.maximum(v, gtm1)
                        out.append((cg, hh))
                    return tuple(out)

                accs = plsc.parallel_loop(0, nv, unroll, carry=accs)(body)
                cnt = accc[rr]
                hht = acch[rr]
                for (cg, hh) in accs:
                    cnt = cnt + cg
                    hht = hht + hh
                accc[rr] = cnt
                acch[rr] = hht
                return 0

            lax.fori_loop(0, RG, row_body, 0)

        def buf_loader(buf, base_col):
            def load(rr, t):
                return buf[rr, pl.ds(t * L, L)], base_col + t * L
            return load

        for k in range(NFULL):
            slot = k % 2
            handles[slot].wait()
            process(buf_loader(bufs[slot], c0 + k * CHUNKC), CHUNKC // L, U)
            if k + 2 < NFULL:
                handles[slot] = issue_full(k + 2, slot)

        @pl.when(h == 0)
        def _():
            pltpu.sync_copy(
                score_hbm.at[pl.ds(g8, RG),
                             pl.ds(pl.multiple_of(NFULL * CHUNKC, 128), T0)],
                tail0)
            process(buf_loader(tail0, NFULL * CHUNKC), T0 // L, UT0)

        @pl.when(h == 1)
        def _():
            pltpu.sync_copy(
                score_hbm.at[pl.ds(g8, RG),
                             pl.ds(pl.multiple_of(SPLIT + NFULL * CHUNKC, 128), T1)],
                tail1)
            process(buf_loader(tail1, SPLIT + NFULL * CHUNKC), T1 // L, UT1)

            def tail_load(rr, t):
                return tbuf[pl.ds(rr * TS + t * L, L)], NT + t * L
            process(tail_load, TS // L, TS // L)

        pack = jnp.zeros((L,), jnp.float32)
        ncols = chw.astype(jnp.float32)
        for rr in range(RG):
            cntf = jnp.max(accc[rr]).astype(jnp.float32)
            gtm1s = jnp.max(gtr[rr]) - 1.0
            hsum = jnp.sum(acch[rr]) - ncols * gtm1s
            pack = (pack + jnp.where(iota == rr, cntf, 0.0)
                    + jnp.where(iota == rr + RG, hsum, 0.0))
        ov[...] = pack
        pltpu.sync_copy(ov, shared.at[s, pl.ds(0, L)])
        plsc.subcore_barrier()
        pltpu.sync_copy(shared.at[s ^ 1, pl.ds(0, L)], pv)
        combined = ov[...] + pv[...]

        ris, rvs, hh2, hoffs = [], [], [], []
        hhandles = []
        for rr in range(RG):
            rv = _gather(combined, jnp.zeros((L,), jnp.int32) + rr)
            hv2 = _gather(combined, jnp.zeros((L,), jnp.int32) + (rr + RG))
            ri = rv.astype(jnp.int32)
            rs = jnp.max(ri)
            hoff = pl.multiple_of(jnp.minimum((rs // 8) * 8, N - L), 8)
            hhandles.append(pltpu.async_copy(
                harm_hbm.at[pl.ds(hoff, L)], hwin.at[rr], semg))
            ris.append(ri)
            rvs.append(rv)
            hh2.append(hv2)
            hoffs.append(hoff)
        for hd in hhandles:
            hd.wait()

        contrib = jnp.zeros((L,), jnp.float32)
        for rr in range(RG):
            hv = _gather(hwin[rr], ris[rr] - hoffs[rr])
            wv = jnp.where(ris[rr] > 0,
                           hv / jnp.maximum(rvs[rr], 1.0), 0.0)
            contrib = contrib + jnp.where(iota == rr,
                                          wv * (hh2[rr] * (1.0 / B)), 0.0)
        contrib = jnp.where(h == 0, contrib, jnp.zeros((L,), jnp.float32))
        ov[...] = contrib
        pltpu.sync_copy(ov, out_hbm.at[c * NS + s])

    return wrpl


_WRPL = _make_kernel()


def kernel(score, target):
    tail = score[:, NT:].reshape(-1)
    parts = _WRPL(score, tail, target, jnp.asarray(_HARM))
    return jnp.sum(parts)
